# baseline (speedup 1.0000x reference)
import functools

import jax
import jax.numpy as jnp
from jax import lax
from jax.experimental import pallas as pl
from jax.experimental.pallas import tpu as pltpu
from jax.experimental.pallas import tpu_sc as plsc

NUM_TOKENS = 16384
HIDDEN = 768
EPS = 1e-7
LANES = 16
NUM_WORKERS = 32
TOK_PER_W = NUM_TOKENS // NUM_WORKERS
CHUNK = 32
NCHUNK = TOK_PER_W // CHUNK
NPAIR = NCHUNK // 2
UNROLL = 8


def _rsqrt(x):
    i = lax.bitcast_convert_type(x, jnp.int32)
    i = jnp.full((LANES,), 0x5F3759DF, jnp.int32) - (i >> 1)
    y = lax.bitcast_convert_type(i, jnp.float32)
    half = x * 0.5
    for _ in range(3):
        y = y * (1.5 - half * y * y)
    return y


def _sc_body(ids_hbm, pids_hbm, word_hbm, pos_hbm, gamma_hbm, beta_hbm,
             out_hbm,
             idx_w0, idx_p0, rows_w0, rows_p0,
             idx_w1, idx_p1, rows_w1, rows_p1,
             vsum, gam_v, bet_v,
             sem_w0, sem_p0, sem_w1, sem_p1):
    wid = lax.axis_index("s") * 2 + lax.axis_index("c")
    base = wid * TOK_PER_W

    pltpu.sync_copy(gamma_hbm, gam_v)
    pltpu.sync_copy(beta_hbm, bet_v)

    bufs = ((idx_w0, idx_p0, rows_w0, rows_p0, sem_w0, sem_p0),
            (idx_w1, idx_p1, rows_w1, rows_p1, sem_w1, sem_p1))

    def start_gather(buf, cbase):
        idx_w, idx_p, rows_w, rows_p, sem_w, sem_p = buf
        pltpu.sync_copy(ids_hbm.at[pl.ds(cbase, CHUNK)], idx_w)
        pltpu.sync_copy(pids_hbm.at[pl.ds(cbase, CHUNK)], idx_p)
        pltpu.make_async_copy(word_hbm.at[idx_w], rows_w, sem_w).start()
        pltpu.make_async_copy(pos_hbm.at[idx_p], rows_p, sem_p).start()

    def wait_gather(buf):
        idx_w, idx_p, rows_w, rows_p, sem_w, sem_p = buf
        pltpu.make_async_copy(word_hbm.at[idx_w], rows_w, sem_w).wait()
        pltpu.make_async_copy(pos_hbm.at[idx_p], rows_p, sem_p).wait()

    def process_chunk(buf):
        rows_w, rows_p = buf[2], buf[3]
        for group in range(CHUNK // LANES):
            row_vec = (jnp.full((LANES,), group * LANES, jnp.int32)
                       + lax.iota(jnp.int32, LANES))

            def p1(g, carry):
                accs = list(carry[:4])
                acc2s = list(carry[4:])
                d0 = g * UNROLL
                for k in range(UNROLL):
                    d = d0 + k
                    col = jnp.full((LANES,), d, jnp.int32)
                    w = plsc.load_gather(rows_w, [row_vec, col])
                    p = plsc.load_gather(rows_p, [row_vec, col])
                    v = w + p
                    vsum[pl.ds(d * LANES, LANES)] = v
                    accs[k % 4] = accs[k % 4] + v
                    acc2s[k % 4] = acc2s[k % 4] + v * v
                return tuple(accs) + tuple(acc2s)

            z = jnp.zeros((LANES,), jnp.float32)
            c = lax.fori_loop(0, HIDDEN // UNROLL, p1, (z,) * 8)
            acc = (c[0] + c[1]) + (c[2] + c[3])
            acc2 = (c[4] + c[5]) + (c[6] + c[7])
            mean = acc * (1.0 / HIDDEN)
            var = acc2 * (1.0 / HIDDEN) - mean * mean
            rstd = _rsqrt(var + EPS)
            mrs = mean * rstd

            def p2(g, _):
                d0 = g * LANES
                gvec = gam_v[pl.ds(d0, LANES)]
                bvec = bet_v[pl.ds(d0, LANES)]
                for k in range(LANES):
                    d = d0 + k
                    v = vsum[pl.ds(d * LANES, LANES)]
                    col = jnp.full((LANES,), d, jnp.int32)
                    plsc.store_scatter(rows_w, [row_vec, col],
                                       (v * rstd - mrs) * gvec[k] + bvec[k])
                return 0

            lax.fori_loop(0, HIDDEN // LANES, p2, 0)

    def writeback(buf, cbase):
        pltpu.sync_copy(buf[2], out_hbm.at[pl.ds(cbase, CHUNK)])

    start_gather(bufs[0], base)

    def pair(h, _):
        c0 = base + (2 * h) * CHUNK
        start_gather(bufs[1], c0 + CHUNK)
        wait_gather(bufs[0])
        process_chunk(bufs[0])
        writeback(bufs[0], c0)

        @pl.when(h < NPAIR - 1)
        def _():
            start_gather(bufs[0], c0 + 2 * CHUNK)

        wait_gather(bufs[1])
        process_chunk(bufs[1])
        writeback(bufs[1], c0 + CHUNK)
        return 0

    lax.fori_loop(0, NPAIR, pair, 0)


def kernel(input_ids, seq_lens, position_ids, word_embeddings,
           position_embeddings, ln_gamma, ln_beta):
    del seq_lens
    mesh = plsc.VectorSubcoreMesh(core_axis_name="c", subcore_axis_name="s")
    k = functools.partial(
        pl.kernel,
        mesh=mesh,
        compiler_params=pltpu.CompilerParams(needs_layout_passes=False),
        out_type=jax.ShapeDtypeStruct((NUM_TOKENS, HIDDEN), jnp.float32),
        scratch_types=[
            pltpu.VMEM((CHUNK,), jnp.int32),
            pltpu.VMEM((CHUNK,), jnp.int32),
            pltpu.VMEM((CHUNK, HIDDEN), jnp.float32),
            pltpu.VMEM((CHUNK, HIDDEN), jnp.float32),
            pltpu.VMEM((CHUNK,), jnp.int32),
            pltpu.VMEM((CHUNK,), jnp.int32),
            pltpu.VMEM((CHUNK, HIDDEN), jnp.float32),
            pltpu.VMEM((CHUNK, HIDDEN), jnp.float32),
            pltpu.VMEM((HIDDEN * LANES,), jnp.float32),
            pltpu.VMEM((HIDDEN,), jnp.float32),
            pltpu.VMEM((HIDDEN,), jnp.float32),
            pltpu.SemaphoreType.DMA,
            pltpu.SemaphoreType.DMA,
            pltpu.SemaphoreType.DMA,
            pltpu.SemaphoreType.DMA,
        ],
    )(_sc_body)
    return k(input_ids.astype(jnp.int32), position_ids.astype(jnp.int32),
             word_embeddings, position_embeddings, ln_gamma, ln_beta)


# speedup vs baseline: 2.3658x; 2.3658x over previous
import functools

import jax
import jax.numpy as jnp
from jax import lax
from jax.experimental import pallas as pl
from jax.experimental.pallas import tpu as pltpu
from jax.experimental.pallas import tpu_sc as plsc

NUM_TOKENS = 16384
HIDDEN = 768
EPS = 1e-7
LANES = 16
NUM_WORKERS = 32
TOK_PER_W = NUM_TOKENS // NUM_WORKERS
CHUNK = 16
NCHUNK = TOK_PER_W // CHUNK
NPAIR = NCHUNK // 2
DVECS = HIDDEN // LANES


def _rsqrt(x):
    i = lax.bitcast_convert_type(x, jnp.int32)
    i = jnp.full((LANES,), 0x5F3759DF, jnp.int32) - (i >> 1)
    y = lax.bitcast_convert_type(i, jnp.float32)
    half = x * 0.5
    for _ in range(3):
        y = y * (1.5 - half * y * y)
    return y


_GATHER_DNUMS = lax.GatherDimensionNumbers(
    offset_dims=(), collapsed_slice_dims=(0,), start_index_map=(0,))


def _allreduce_sum(v):
    lane = lax.iota(jnp.int32, LANES)
    for shift in (8, 4, 2, 1):
        idx = lane ^ shift
        v = v + lax.gather(v, idx[:, None], _GATHER_DNUMS, (1,),
                           mode=lax.GatherScatterMode.PROMISE_IN_BOUNDS)
    return v


def _sc_body(ids_hbm, pids_hbm, word_hbm, pos_hbm, gamma_hbm, beta_hbm,
             out_hbm,
             idx_w0, idx_p0, rows_w0, rows_p0,
             idx_w1, idx_p1, rows_w1, rows_p1,
             out0, out1, gam_v, bet_v,
             sem_w0, sem_p0, sem_w1, sem_p1, sem_o0, sem_o1):
    wid = lax.axis_index("s") * 2 + lax.axis_index("c")
    base = wid * TOK_PER_W

    pltpu.sync_copy(gamma_hbm, gam_v)
    pltpu.sync_copy(beta_hbm, bet_v)

    bufs = ((idx_w0, idx_p0, rows_w0, rows_p0, out0, sem_w0, sem_p0, sem_o0),
            (idx_w1, idx_p1, rows_w1, rows_p1, out1, sem_w1, sem_p1, sem_o1))

    def start_gather(buf, cbase):
        idx_w, idx_p, rows_w, rows_p = buf[0], buf[1], buf[2], buf[3]
        pltpu.sync_copy(ids_hbm.at[pl.ds(cbase, CHUNK)], idx_w)
        pltpu.sync_copy(pids_hbm.at[pl.ds(cbase, CHUNK)], idx_p)
        pltpu.make_async_copy(word_hbm.at[idx_w], rows_w, buf[5]).start()
        pltpu.make_async_copy(pos_hbm.at[idx_p], rows_p, buf[6]).start()

    def wait_gather(buf):
        pltpu.make_async_copy(word_hbm.at[buf[0]], buf[2], buf[5]).wait()
        pltpu.make_async_copy(pos_hbm.at[buf[1]], buf[3], buf[6]).wait()

    def start_wb(buf, cbase):
        pltpu.make_async_copy(buf[4], out_hbm.at[pl.ds(cbase, CHUNK)],
                              buf[7]).start()

    def wait_wb(buf, cbase):
        pltpu.make_async_copy(buf[4], out_hbm.at[pl.ds(cbase, CHUNK)],
                              buf[7]).wait()

    def process_chunk(buf):
        rows_w, rows_p, out_v = buf[2], buf[3], buf[4]

        def two_rows(r2, _):
            rows = (r2 * 2, r2 * 2 + 1)
            za = [jnp.zeros((LANES,), jnp.float32)] * 4
            accs = [list(za), list(za)]
            acc2s = [list(za), list(za)]
            for j in range(DVECS):
                sl = pl.ds(j * LANES, LANES)
                for i, r in enumerate(rows):
                    v = rows_w[r, sl] + rows_p[r, sl]
                    out_v[r, sl] = v
                    accs[i][j % 4] = accs[i][j % 4] + v
                    acc2s[i][j % 4] = acc2s[i][j % 4] + v * v
            stats = []
            for i in range(2):
                a = accs[i]
                a2 = acc2s[i]
                acc = (a[0] + a[1]) + (a[2] + a[3])
                acc2 = (a2[0] + a2[1]) + (a2[2] + a2[3])
                mean = _allreduce_sum(acc) * (1.0 / HIDDEN)
                var = _allreduce_sum(acc2) * (1.0 / HIDDEN) - mean * mean
                rstd = _rsqrt(var + EPS)
                stats.append((rstd, mean * rstd))
            for j in range(DVECS):
                sl = pl.ds(j * LANES, LANES)
                g = gam_v[sl]
                b = bet_v[sl]
                for i, r in enumerate(rows):
                    rstd, mrs = stats[i]
                    out_v[r, sl] = (out_v[r, sl] * rstd - mrs) * g + b
            return 0

        lax.fori_loop(0, CHUNK // 2, two_rows, 0)

    start_gather(bufs[0], base)

    def pair(h, _):
        c0 = base + (2 * h) * CHUNK
        c1 = c0 + CHUNK
        start_gather(bufs[1], c1)

        @pl.when(h > 0)
        def _():
            wait_wb(bufs[0], c0 - 2 * CHUNK)

        wait_gather(bufs[0])
        process_chunk(bufs[0])
        start_wb(bufs[0], c0)

        @pl.when(h < NPAIR - 1)
        def _():
            start_gather(bufs[0], c0 + 2 * CHUNK)

        @pl.when(h > 0)
        def _():
            wait_wb(bufs[1], c1 - 2 * CHUNK)

        wait_gather(bufs[1])
        process_chunk(bufs[1])
        start_wb(bufs[1], c1)
        return 0

    lax.fori_loop(0, NPAIR, pair, 0)

    last0 = base + (NCHUNK - 2) * CHUNK
    wait_wb(bufs[0], last0)
    wait_wb(bufs[1], last0 + CHUNK)


def kernel(input_ids, seq_lens, position_ids, word_embeddings,
           position_embeddings, ln_gamma, ln_beta):
    del seq_lens
    mesh = plsc.VectorSubcoreMesh(core_axis_name="c", subcore_axis_name="s")
    k = functools.partial(
        pl.kernel,
        mesh=mesh,
        out_type=jax.ShapeDtypeStruct((NUM_TOKENS, HIDDEN), jnp.float32),
        scratch_types=[
            pltpu.VMEM((CHUNK,), jnp.int32),
            pltpu.VMEM((CHUNK,), jnp.int32),
            pltpu.VMEM((CHUNK, HIDDEN), jnp.float32),
            pltpu.VMEM((CHUNK, HIDDEN), jnp.float32),
            pltpu.VMEM((CHUNK,), jnp.int32),
            pltpu.VMEM((CHUNK,), jnp.int32),
            pltpu.VMEM((CHUNK, HIDDEN), jnp.float32),
            pltpu.VMEM((CHUNK, HIDDEN), jnp.float32),
            pltpu.VMEM((CHUNK, HIDDEN), jnp.float32),
            pltpu.VMEM((CHUNK, HIDDEN), jnp.float32),
            pltpu.VMEM((HIDDEN,), jnp.float32),
            pltpu.VMEM((HIDDEN,), jnp.float32),
            pltpu.SemaphoreType.DMA,
            pltpu.SemaphoreType.DMA,
            pltpu.SemaphoreType.DMA,
            pltpu.SemaphoreType.DMA,
            pltpu.SemaphoreType.DMA,
            pltpu.SemaphoreType.DMA,
        ],
    )(_sc_body)
    return k(input_ids.astype(jnp.int32), position_ids.astype(jnp.int32),
             word_embeddings, position_embeddings, ln_gamma, ln_beta)


# speedup vs baseline: 3.8386x; 1.6225x over previous
import functools

import jax
import jax.numpy as jnp
from jax import lax
from jax.experimental import pallas as pl
from jax.experimental.pallas import tpu as pltpu
from jax.experimental.pallas import tpu_sc as plsc

NUM_TOKENS = 16384
HIDDEN = 768
EPS = 1e-7
LANES = 16
NUM_WORKERS = 32
TOK_PER_W = NUM_TOKENS // NUM_WORKERS
CHUNK = 16
NCHUNK = TOK_PER_W // CHUNK
NPAIR = NCHUNK // 2
DVECS = HIDDEN // LANES


def _rsqrt(x):
    i = lax.bitcast_convert_type(x, jnp.int32)
    i = jnp.full((LANES,), 0x5F3759DF, jnp.int32) - (i >> 1)
    y = lax.bitcast_convert_type(i, jnp.float32)
    half = x * 0.5
    for _ in range(3):
        y = y * (1.5 - half * y * y)
    return y


_GATHER_DNUMS = lax.GatherDimensionNumbers(
    offset_dims=(), collapsed_slice_dims=(0,), start_index_map=(0,))


def _allreduce_sum(v):
    lane = lax.iota(jnp.int32, LANES)
    for shift in (8, 4, 2, 1):
        idx = lane ^ shift
        v = v + lax.gather(v, idx[:, None], _GATHER_DNUMS, (1,),
                           mode=lax.GatherScatterMode.PROMISE_IN_BOUNDS)
    return v


def _sc_body(ids_hbm, pids_hbm, word_hbm, pos_hbm, gamma_hbm, beta_hbm,
             out_hbm,
             idx_w0, idx_p0, rows_w0, rows_p0,
             idx_w1, idx_p1, rows_w1, rows_p1,
             out0, out1, gam_v, bet_v,
             sem_w0, sem_p0, sem_w1, sem_p1, sem_o0, sem_o1):
    wid = lax.axis_index("s") * 2 + lax.axis_index("c")
    base = wid * TOK_PER_W

    pltpu.sync_copy(gamma_hbm, gam_v)
    pltpu.sync_copy(beta_hbm, bet_v)

    bufs = ((idx_w0, idx_p0, rows_w0, rows_p0, out0, sem_w0, sem_p0, sem_o0),
            (idx_w1, idx_p1, rows_w1, rows_p1, out1, sem_w1, sem_p1, sem_o1))

    def start_gather(buf, cbase):
        idx_w, idx_p, rows_w, rows_p = buf[0], buf[1], buf[2], buf[3]
        pltpu.sync_copy(ids_hbm.at[pl.ds(cbase, CHUNK)], idx_w)
        pltpu.sync_copy(pids_hbm.at[pl.ds(cbase, CHUNK)], idx_p)
        pltpu.make_async_copy(word_hbm.at[idx_w], rows_w, buf[5]).start()
        pltpu.make_async_copy(pos_hbm.at[idx_p], rows_p, buf[6]).start()

    def wait_gather(buf):
        pltpu.make_async_copy(word_hbm.at[buf[0]], buf[2], buf[5]).wait()
        pltpu.make_async_copy(pos_hbm.at[buf[1]], buf[3], buf[6]).wait()

    def start_wb(buf, cbase):
        pltpu.make_async_copy(buf[4], out_hbm.at[pl.ds(cbase, CHUNK)],
                              buf[7]).start()

    def wait_wb(buf, cbase):
        pltpu.make_async_copy(buf[4], out_hbm.at[pl.ds(cbase, CHUNK)],
                              buf[7]).wait()

    def process_chunk(buf):
        rows_w, rows_p, out_v = buf[2], buf[3], buf[4]

        GRP = 8 * LANES

        def one_row(r, _):
            z = jnp.zeros((LANES,), jnp.float32)

            @plsc.parallel_loop(0, HIDDEN, GRP, unroll=2, carry=(z,) * 8)
            def acc_out(d, carry):
                a = list(carry[:4])
                a2 = list(carry[4:])
                for k in range(8):
                    sl = pl.ds(d + k * LANES, LANES)
                    v = rows_w[r, sl] + rows_p[r, sl]
                    out_v[r, sl] = v
                    a[k % 4] = a[k % 4] + v
                    a2[k % 4] = a2[k % 4] + v * v
                return tuple(a) + tuple(a2)

            c = acc_out
            acc = (c[0] + c[1]) + (c[2] + c[3])
            acc2 = (c[4] + c[5]) + (c[6] + c[7])
            mean = _allreduce_sum(acc) * (1.0 / HIDDEN)
            var = _allreduce_sum(acc2) * (1.0 / HIDDEN) - mean * mean
            rstd = _rsqrt(var + EPS)
            mrs = mean * rstd

            @plsc.parallel_loop(0, HIDDEN, GRP, unroll=2)
            def norm(d):
                for k in range(8):
                    sl = pl.ds(d + k * LANES, LANES)
                    out_v[r, sl] = ((out_v[r, sl] * rstd - mrs) * gam_v[sl]
                                    + bet_v[sl])

            return 0

        lax.fori_loop(0, CHUNK, one_row, 0)

    start_gather(bufs[0], base)

    def pair(h, _):
        c0 = base + (2 * h) * CHUNK
        c1 = c0 + CHUNK
        start_gather(bufs[1], c1)

        @pl.when(h > 0)
        def _():
            wait_wb(bufs[0], c0 - 2 * CHUNK)

        wait_gather(bufs[0])
        process_chunk(bufs[0])
        start_wb(bufs[0], c0)

        @pl.when(h < NPAIR - 1)
        def _():
            start_gather(bufs[0], c0 + 2 * CHUNK)

        @pl.when(h > 0)
        def _():
            wait_wb(bufs[1], c1 - 2 * CHUNK)

        wait_gather(bufs[1])
        process_chunk(bufs[1])
        start_wb(bufs[1], c1)
        return 0

    lax.fori_loop(0, NPAIR, pair, 0)

    last0 = base + (NCHUNK - 2) * CHUNK
    wait_wb(bufs[0], last0)
    wait_wb(bufs[1], last0 + CHUNK)


def kernel(input_ids, seq_lens, position_ids, word_embeddings,
           position_embeddings, ln_gamma, ln_beta):
    del seq_lens
    mesh = plsc.VectorSubcoreMesh(core_axis_name="c", subcore_axis_name="s")
    k = functools.partial(
        pl.kernel,
        mesh=mesh,
        out_type=jax.ShapeDtypeStruct((NUM_TOKENS, HIDDEN), jnp.float32),
        scratch_types=[
            pltpu.VMEM((CHUNK,), jnp.int32),
            pltpu.VMEM((CHUNK,), jnp.int32),
            pltpu.VMEM((CHUNK, HIDDEN), jnp.float32),
            pltpu.VMEM((CHUNK, HIDDEN), jnp.float32),
            pltpu.VMEM((CHUNK,), jnp.int32),
            pltpu.VMEM((CHUNK,), jnp.int32),
            pltpu.VMEM((CHUNK, HIDDEN), jnp.float32),
            pltpu.VMEM((CHUNK, HIDDEN), jnp.float32),
            pltpu.VMEM((CHUNK, HIDDEN), jnp.float32),
            pltpu.VMEM((CHUNK, HIDDEN), jnp.float32),
            pltpu.VMEM((HIDDEN,), jnp.float32),
            pltpu.VMEM((HIDDEN,), jnp.float32),
            pltpu.SemaphoreType.DMA,
            pltpu.SemaphoreType.DMA,
            pltpu.SemaphoreType.DMA,
            pltpu.SemaphoreType.DMA,
            pltpu.SemaphoreType.DMA,
            pltpu.SemaphoreType.DMA,
        ],
    )(_sc_body)
    return k(input_ids.astype(jnp.int32), position_ids.astype(jnp.int32),
             word_embeddings, position_embeddings, ln_gamma, ln_beta)
